# baseline (speedup 1.0000x reference)
import math

import jax
import jax.numpy as jnp
from jax.experimental import pallas as pl
from jax.experimental.pallas import tpu as pltpu

_N = 8192
_D = 16
_BM = 1024
_BK = 2048
_BK1 = 1024


def _body(coef_ref, a_tile, p_tile, u_ref, s_ref,
          u16_ref, s16_ref, s32_ref, acc_u, acc_s, nj):
    j = pl.program_id(1)

    @pl.when(j == 0)
    def _():
        acc_u[...] = jnp.zeros_like(acc_u)
        acc_s[...] = jnp.zeros_like(acc_s)

    acc_u[...] += jnp.dot(a_tile, u_ref[...],
                          preferred_element_type=jnp.float32)
    acc_s[...] += jnp.dot(p_tile, s_ref[...],
                          preferred_element_type=jnp.float32)

    @pl.when(j == nj - 1)
    def _():
        au = acc_u[...]
        s_new = coef_ref[0, 0] * au + acc_s[...]
        u16_ref[...] = au.astype(jnp.bfloat16)
        s16_ref[...] = s_new.astype(jnp.bfloat16)
        s32_ref[...] = s_new


def _step_kern(coef_ref, a_ref, p_ref, u_ref, s_ref,
               u16_ref, s16_ref, s32_ref, acc_u, acc_s):
    _body(coef_ref, a_ref[...], p_ref[...], u_ref, s_ref,
          u16_ref, s16_ref, s32_ref, acc_u, acc_s, _N // _BK)


def _step1_kern(coef_ref, a_ref, p_ref, u_ref, s_ref,
                u16_ref, s16_ref, s32_ref, a16_ref, p16_ref, acc_u, acc_s):
    a16 = a_ref[...].astype(jnp.bfloat16)
    p16 = p_ref[...].astype(jnp.bfloat16)
    a16_ref[...] = a16
    p16_ref[...] = p16
    _body(coef_ref, a16, p16, u_ref, s_ref,
          u16_ref, s16_ref, s32_ref, acc_u, acc_s, _N // _BK1)


def _vec_specs():
    return [
        pl.BlockSpec((_BM, _D), lambda i, j: (i, 0)),
        pl.BlockSpec((_BM, _D), lambda i, j: (i, 0)),
        pl.BlockSpec((_BM, _D), lambda i, j: (i, 0)),
    ]


def _vec_shapes():
    return [
        jax.ShapeDtypeStruct((_N, _D), jnp.bfloat16),
        jax.ShapeDtypeStruct((_N, _D), jnp.bfloat16),
        jax.ShapeDtypeStruct((_N, _D), jnp.float32),
    ]


_step = pl.pallas_call(
    _step_kern,
    grid=(_N // _BM, _N // _BK),
    in_specs=[
        pl.BlockSpec(memory_space=pltpu.SMEM),
        pl.BlockSpec((_BM, _BK), lambda i, j: (i, j)),
        pl.BlockSpec((_BM, _BK), lambda i, j: (i, j)),
        pl.BlockSpec((_BK, _D), lambda i, j: (j, 0)),
        pl.BlockSpec((_BK, _D), lambda i, j: (j, 0)),
    ],
    out_specs=_vec_specs(),
    out_shape=_vec_shapes(),
    scratch_shapes=[
        pltpu.VMEM((_BM, _D), jnp.float32),
        pltpu.VMEM((_BM, _D), jnp.float32),
    ],
    compiler_params=pltpu.CompilerParams(
        dimension_semantics=("parallel", "arbitrary"),
    ),
)

_step1 = pl.pallas_call(
    _step1_kern,
    grid=(_N // _BM, _N // _BK1),
    in_specs=[
        pl.BlockSpec(memory_space=pltpu.SMEM),
        pl.BlockSpec((_BM, _BK1), lambda i, j: (i, j)),
        pl.BlockSpec((_BM, _BK1), lambda i, j: (i, j)),
        pl.BlockSpec((_BK1, _D), lambda i, j: (j, 0)),
        pl.BlockSpec((_BK1, _D), lambda i, j: (j, 0)),
    ],
    out_specs=_vec_specs() + [
        pl.BlockSpec((_BM, _BK1), lambda i, j: (i, j)),
        pl.BlockSpec((_BM, _BK1), lambda i, j: (i, j)),
    ],
    out_shape=_vec_shapes() + [
        jax.ShapeDtypeStruct((_N, _N), jnp.bfloat16),
        jax.ShapeDtypeStruct((_N, _N), jnp.bfloat16),
    ],
    scratch_shapes=[
        pltpu.VMEM((_BM, _D), jnp.float32),
        pltpu.VMEM((_BM, _D), jnp.float32),
    ],
    compiler_params=pltpu.CompilerParams(
        dimension_semantics=("parallel", "arbitrary"),
    ),
)


def kernel(x, adj, poly_item, filter_param):
    k = filter_param.shape[0] - 1
    fp = jax.nn.relu(filter_param)[:, 0]
    coefs = [math.comb(k, i) / (2.0 ** k) for i in range(k + 1)]
    u16 = x.astype(jnp.bfloat16)
    s16 = (coefs[k] * fp[k] * x).astype(jnp.bfloat16)
    c = (coefs[k - 1] * fp[k - 1]).reshape(1, 1)
    u16, s16, s32, a16, p16 = _step1(c, adj, poly_item, u16, s16)
    for t in range(2, k + 1):
        c = (coefs[k - t] * fp[k - t]).reshape(1, 1)
        u16, s16, s32 = _step(c, a16, p16, u16, s16)
    return s32


# speedup vs baseline: 1.0755x; 1.0755x over previous
import math

import jax
import jax.numpy as jnp
from jax.experimental import pallas as pl
from jax.experimental.pallas import tpu as pltpu

_N = 8192
_D = 16
_BM = 1024
_BK = 2048
_BK1 = 1024


def _step_kern(a_ref, p_ref, u_ref, s_ref, au_ref, ps_ref):
    j = pl.program_id(1)

    @pl.when(j == 0)
    def _():
        au_ref[...] = jnp.zeros_like(au_ref)
        ps_ref[...] = jnp.zeros_like(ps_ref)

    u_blk = u_ref[pl.ds(j * _BK, _BK), :]
    s_blk = s_ref[pl.ds(j * _BK, _BK), :]
    au_ref[...] += jnp.dot(a_ref[...], u_blk,
                           preferred_element_type=jnp.float32)
    ps_ref[...] += jnp.dot(p_ref[...], s_blk,
                           preferred_element_type=jnp.float32)


def _step1_kern(a_ref, p_ref, u_ref, s_ref, au_ref, ps_ref, a16_ref, p16_ref):
    j = pl.program_id(1)

    @pl.when(j == 0)
    def _():
        au_ref[...] = jnp.zeros_like(au_ref)
        ps_ref[...] = jnp.zeros_like(ps_ref)

    a16 = a_ref[...].astype(jnp.bfloat16)
    p16 = p_ref[...].astype(jnp.bfloat16)
    a16_ref[...] = a16
    p16_ref[...] = p16
    u_blk = u_ref[pl.ds(j * _BK1, _BK1), :]
    s_blk = s_ref[pl.ds(j * _BK1, _BK1), :]
    au_ref[...] += jnp.dot(a16, u_blk,
                           preferred_element_type=jnp.float32)
    ps_ref[...] += jnp.dot(p16, s_blk,
                           preferred_element_type=jnp.float32)


_step = pl.pallas_call(
    _step_kern,
    grid=(_N // _BM, _N // _BK),
    in_specs=[
        pl.BlockSpec((_BM, _BK), lambda i, j: (i, j)),
        pl.BlockSpec((_BM, _BK), lambda i, j: (i, j)),
        pl.BlockSpec((_N, _D), lambda i, j: (0, 0)),
        pl.BlockSpec((_N, _D), lambda i, j: (0, 0)),
    ],
    out_specs=[
        pl.BlockSpec((_BM, _D), lambda i, j: (i, 0)),
        pl.BlockSpec((_BM, _D), lambda i, j: (i, 0)),
    ],
    out_shape=[
        jax.ShapeDtypeStruct((_N, _D), jnp.float32),
        jax.ShapeDtypeStruct((_N, _D), jnp.float32),
    ],
    compiler_params=pltpu.CompilerParams(
        dimension_semantics=("parallel", "arbitrary"),
    ),
)

_step1 = pl.pallas_call(
    _step1_kern,
    grid=(_N // _BM, _N // _BK1),
    in_specs=[
        pl.BlockSpec((_BM, _BK1), lambda i, j: (i, j)),
        pl.BlockSpec((_BM, _BK1), lambda i, j: (i, j)),
        pl.BlockSpec((_N, _D), lambda i, j: (0, 0)),
        pl.BlockSpec((_N, _D), lambda i, j: (0, 0)),
    ],
    out_specs=[
        pl.BlockSpec((_BM, _D), lambda i, j: (i, 0)),
        pl.BlockSpec((_BM, _D), lambda i, j: (i, 0)),
        pl.BlockSpec((_BM, _BK1), lambda i, j: (i, j)),
        pl.BlockSpec((_BM, _BK1), lambda i, j: (i, j)),
    ],
    out_shape=[
        jax.ShapeDtypeStruct((_N, _D), jnp.float32),
        jax.ShapeDtypeStruct((_N, _D), jnp.float32),
        jax.ShapeDtypeStruct((_N, _N), jnp.bfloat16),
        jax.ShapeDtypeStruct((_N, _N), jnp.bfloat16),
    ],
    compiler_params=pltpu.CompilerParams(
        dimension_semantics=("parallel", "arbitrary"),
    ),
)


def kernel(x, adj, poly_item, filter_param):
    k = filter_param.shape[0] - 1
    fp = jax.nn.relu(filter_param)[:, 0]
    coefs = [math.comb(k, i) / (2.0 ** k) for i in range(k + 1)]
    u = x
    s = coefs[k] * fp[k] * x
    au, ps, a16, p16 = _step1(adj, poly_item,
                              u.astype(jnp.bfloat16), s.astype(jnp.bfloat16))
    u = au
    s = coefs[k - 1] * fp[k - 1] * au + ps
    for t in range(2, k + 1):
        au, ps = _step(a16, p16,
                       u.astype(jnp.bfloat16), s.astype(jnp.bfloat16))
        u = au
        s = coefs[k - t] * fp[k - t] * au + ps
    return s


# speedup vs baseline: 1.1113x; 1.0333x over previous
import math

import jax
import jax.numpy as jnp
from jax.experimental import pallas as pl
from jax.experimental.pallas import tpu as pltpu

_N = 8192
_D = 16
_BM = 1024
_BK = 2048
_BK1 = 1024
_NT = 3


def _step1_kern(a_ref, p_ref, u_ref, s_ref, au_ref, ps_ref, a16_ref, p16_ref):
    j = pl.program_id(1)

    @pl.when(j == 0)
    def _():
        au_ref[...] = jnp.zeros_like(au_ref)
        ps_ref[...] = jnp.zeros_like(ps_ref)

    a16 = a_ref[...].astype(jnp.bfloat16)
    p16 = p_ref[...].astype(jnp.bfloat16)
    a16_ref[...] = a16
    p16_ref[...] = p16
    u_blk = u_ref[pl.ds(j * _BK1, _BK1), :]
    s_blk = s_ref[pl.ds(j * _BK1, _BK1), :]
    au_ref[...] += jnp.dot(a16, u_blk,
                           preferred_element_type=jnp.float32)
    ps_ref[...] += jnp.dot(p16, s_blk,
                           preferred_element_type=jnp.float32)


def _steps_kern(coef_ref, a_ref, p_ref, u0_ref, s0_ref, y_ref,
                u_scr, s_scr, acc_u, acc_s):
    t = pl.program_id(0)
    i = pl.program_id(1)
    j = pl.program_id(2)
    nj = _N // _BK

    @pl.when(j == 0)
    def _():
        acc_u[...] = jnp.zeros_like(acc_u)
        acc_s[...] = jnp.zeros_like(acc_s)

    rslot = jax.lax.rem(t + 1, 2)
    wslot = jax.lax.rem(t, 2)
    base = j * _BK
    first = (t == 0)
    u_blk = jnp.where(first, u0_ref[pl.ds(base, _BK), :],
                      u_scr[rslot, pl.ds(base, _BK), :])
    s_blk = jnp.where(first, s0_ref[pl.ds(base, _BK), :],
                      s_scr[rslot, pl.ds(base, _BK), :])
    acc_u[...] += jnp.dot(a_ref[...], u_blk,
                          preferred_element_type=jnp.float32)
    acc_s[...] += jnp.dot(p_ref[...], s_blk,
                          preferred_element_type=jnp.float32)

    @pl.when(j == nj - 1)
    def _():
        au = acc_u[...]
        s_new = coef_ref[t] * au + acc_s[...]
        obase = i * _BM
        u_scr[wslot, pl.ds(obase, _BM), :] = au.astype(jnp.bfloat16)
        s_scr[wslot, pl.ds(obase, _BM), :] = s_new.astype(jnp.bfloat16)

        @pl.when(t == _NT - 1)
        def _():
            y_ref[...] = s_new


_step1 = pl.pallas_call(
    _step1_kern,
    grid=(_N // _BM, _N // _BK1),
    in_specs=[
        pl.BlockSpec((_BM, _BK1), lambda i, j: (i, j)),
        pl.BlockSpec((_BM, _BK1), lambda i, j: (i, j)),
        pl.BlockSpec((_N, _D), lambda i, j: (0, 0)),
        pl.BlockSpec((_N, _D), lambda i, j: (0, 0)),
    ],
    out_specs=[
        pl.BlockSpec((_BM, _D), lambda i, j: (i, 0)),
        pl.BlockSpec((_BM, _D), lambda i, j: (i, 0)),
        pl.BlockSpec((_BM, _BK1), lambda i, j: (i, j)),
        pl.BlockSpec((_BM, _BK1), lambda i, j: (i, j)),
    ],
    out_shape=[
        jax.ShapeDtypeStruct((_N, _D), jnp.float32),
        jax.ShapeDtypeStruct((_N, _D), jnp.float32),
        jax.ShapeDtypeStruct((_N, _N), jnp.bfloat16),
        jax.ShapeDtypeStruct((_N, _N), jnp.bfloat16),
    ],
    compiler_params=pltpu.CompilerParams(
        dimension_semantics=("parallel", "arbitrary"),
    ),
)

_steps234 = pl.pallas_call(
    _steps_kern,
    grid=(_NT, _N // _BM, _N // _BK),
    in_specs=[
        pl.BlockSpec(memory_space=pltpu.SMEM),
        pl.BlockSpec((_BM, _BK), lambda t, i, j: (i, j)),
        pl.BlockSpec((_BM, _BK), lambda t, i, j: (i, j)),
        pl.BlockSpec((_N, _D), lambda t, i, j: (0, 0)),
        pl.BlockSpec((_N, _D), lambda t, i, j: (0, 0)),
    ],
    out_specs=pl.BlockSpec((_BM, _D), lambda t, i, j: (i, 0)),
    out_shape=jax.ShapeDtypeStruct((_N, _D), jnp.float32),
    scratch_shapes=[
        pltpu.VMEM((2, _N, _D), jnp.bfloat16),
        pltpu.VMEM((2, _N, _D), jnp.bfloat16),
        pltpu.VMEM((_BM, _D), jnp.float32),
        pltpu.VMEM((_BM, _D), jnp.float32),
    ],
    compiler_params=pltpu.CompilerParams(
        dimension_semantics=("arbitrary", "arbitrary", "arbitrary"),
    ),
)


def kernel(x, adj, poly_item, filter_param):
    k = filter_param.shape[0] - 1
    fp = jax.nn.relu(filter_param)[:, 0]
    coefs = [math.comb(k, i) / (2.0 ** k) for i in range(k + 1)]
    u = x
    s = coefs[k] * fp[k] * x
    au, ps, a16, p16 = _step1(adj, poly_item,
                              u.astype(jnp.bfloat16), s.astype(jnp.bfloat16))
    u16 = au.astype(jnp.bfloat16)
    s16 = (coefs[k - 1] * fp[k - 1] * au + ps).astype(jnp.bfloat16)
    cvec = jnp.stack([coefs[2] * fp[2], coefs[1] * fp[1], coefs[0] * fp[0]])
    y = _steps234(cvec, a16, p16, u16, s16)
    return y


# speedup vs baseline: 1.1809x; 1.0627x over previous
import math

import jax
import jax.numpy as jnp
from jax.experimental import pallas as pl
from jax.experimental.pallas import tpu as pltpu

_N = 8192
_D = 16
_BM = 1024
_BK = 4096
_BK1 = 2048
_NT = 3


def _step1_kern(a_ref, p_ref, u_ref, s_ref, au_ref, ps_ref, a16_ref, p16_ref):
    j = pl.program_id(1)

    @pl.when(j == 0)
    def _():
        au_ref[...] = jnp.zeros_like(au_ref)
        ps_ref[...] = jnp.zeros_like(ps_ref)

    a16 = a_ref[...].astype(jnp.bfloat16)
    p16 = p_ref[...].astype(jnp.bfloat16)
    a16_ref[...] = a16
    p16_ref[...] = p16
    u_blk = u_ref[pl.ds(j * _BK1, _BK1), :]
    s_blk = s_ref[pl.ds(j * _BK1, _BK1), :]
    au_ref[...] += jnp.dot(a16, u_blk,
                           preferred_element_type=jnp.float32)
    ps_ref[...] += jnp.dot(p16, s_blk,
                           preferred_element_type=jnp.float32)


def _steps_kern(coef_ref, a_ref, p_ref, u0_ref, s0_ref, y_ref,
                u_scr, s_scr, acc_u, acc_s):
    t = pl.program_id(0)
    i = pl.program_id(1)
    j = pl.program_id(2)
    nj = _N // _BK

    @pl.when(j == 0)
    def _():
        acc_u[...] = jnp.zeros_like(acc_u)
        acc_s[...] = jnp.zeros_like(acc_s)

    rslot = jax.lax.rem(t + 1, 2)
    wslot = jax.lax.rem(t, 2)
    base = j * _BK
    first = (t == 0)
    u_blk = jnp.where(first, u0_ref[pl.ds(base, _BK), :],
                      u_scr[rslot, pl.ds(base, _BK), :])
    s_blk = jnp.where(first, s0_ref[pl.ds(base, _BK), :],
                      s_scr[rslot, pl.ds(base, _BK), :])
    acc_u[...] += jnp.dot(a_ref[...], u_blk,
                          preferred_element_type=jnp.float32)
    acc_s[...] += jnp.dot(p_ref[...], s_blk,
                          preferred_element_type=jnp.float32)

    @pl.when(j == nj - 1)
    def _():
        au = acc_u[...]
        s_new = coef_ref[t] * au + acc_s[...]
        obase = i * _BM
        u_scr[wslot, pl.ds(obase, _BM), :] = au.astype(jnp.bfloat16)
        s_scr[wslot, pl.ds(obase, _BM), :] = s_new.astype(jnp.bfloat16)

        @pl.when(t == _NT - 1)
        def _():
            y_ref[...] = s_new


_step1 = pl.pallas_call(
    _step1_kern,
    grid=(_N // _BM, _N // _BK1),
    in_specs=[
        pl.BlockSpec((_BM, _BK1), lambda i, j: (i, j)),
        pl.BlockSpec((_BM, _BK1), lambda i, j: (i, j)),
        pl.BlockSpec((_N, _D), lambda i, j: (0, 0)),
        pl.BlockSpec((_N, _D), lambda i, j: (0, 0)),
    ],
    out_specs=[
        pl.BlockSpec((_BM, _D), lambda i, j: (i, 0)),
        pl.BlockSpec((_BM, _D), lambda i, j: (i, 0)),
        pl.BlockSpec((_BM, _BK1), lambda i, j: (i, j)),
        pl.BlockSpec((_BM, _BK1), lambda i, j: (i, j)),
    ],
    out_shape=[
        jax.ShapeDtypeStruct((_N, _D), jnp.float32),
        jax.ShapeDtypeStruct((_N, _D), jnp.float32),
        jax.ShapeDtypeStruct((_N, _N), jnp.bfloat16),
        jax.ShapeDtypeStruct((_N, _N), jnp.bfloat16),
    ],
    compiler_params=pltpu.CompilerParams(
        dimension_semantics=("parallel", "arbitrary"),
    ),
)

_steps234 = pl.pallas_call(
    _steps_kern,
    grid=(_NT, _N // _BM, _N // _BK),
    in_specs=[
        pl.BlockSpec(memory_space=pltpu.SMEM),
        pl.BlockSpec((_BM, _BK), lambda t, i, j: (i, j)),
        pl.BlockSpec((_BM, _BK), lambda t, i, j: (i, j)),
        pl.BlockSpec((_N, _D), lambda t, i, j: (0, 0)),
        pl.BlockSpec((_N, _D), lambda t, i, j: (0, 0)),
    ],
    out_specs=pl.BlockSpec((_BM, _D), lambda t, i, j: (i, 0)),
    out_shape=jax.ShapeDtypeStruct((_N, _D), jnp.float32),
    scratch_shapes=[
        pltpu.VMEM((2, _N, _D), jnp.bfloat16),
        pltpu.VMEM((2, _N, _D), jnp.bfloat16),
        pltpu.VMEM((_BM, _D), jnp.float32),
        pltpu.VMEM((_BM, _D), jnp.float32),
    ],
    compiler_params=pltpu.CompilerParams(
        dimension_semantics=("arbitrary", "arbitrary", "arbitrary"),
    ),
)


def kernel(x, adj, poly_item, filter_param):
    k = filter_param.shape[0] - 1
    fp = jax.nn.relu(filter_param)[:, 0]
    coefs = [math.comb(k, i) / (2.0 ** k) for i in range(k + 1)]
    u = x
    s = coefs[k] * fp[k] * x
    au, ps, a16, p16 = _step1(adj, poly_item,
                              u.astype(jnp.bfloat16), s.astype(jnp.bfloat16))
    u16 = au.astype(jnp.bfloat16)
    s16 = (coefs[k - 1] * fp[k - 1] * au + ps).astype(jnp.bfloat16)
    cvec = jnp.stack([coefs[2] * fp[2], coefs[1] * fp[1], coefs[0] * fp[0]])
    y = _steps234(cvec, a16, p16, u16, s16)
    return y


# speedup vs baseline: 1.1982x; 1.0146x over previous
import math

import jax
import jax.numpy as jnp
from jax.experimental import pallas as pl
from jax.experimental.pallas import tpu as pltpu

_N = 8192
_D = 16
_BM = 512
_BM1 = 256
_NT = 3


def _step1_kern(a_ref, p_ref, u_ref, s_ref, au_ref, ps_ref, a16_ref, p16_ref):
    a16 = a_ref[...].astype(jnp.bfloat16)
    p16 = p_ref[...].astype(jnp.bfloat16)
    a16_ref[...] = a16
    p16_ref[...] = p16
    au_ref[...] = jnp.dot(a16, u_ref[...],
                          preferred_element_type=jnp.float32)
    ps_ref[...] = jnp.dot(p16, s_ref[...],
                          preferred_element_type=jnp.float32)


def _steps_kern(coef_ref, a_ref, p_ref, u0_ref, s0_ref, y_ref, u_scr, s_scr):
    t = pl.program_id(0)
    i = pl.program_id(1)

    rslot = jax.lax.rem(t + 1, 2)
    wslot = jax.lax.rem(t, 2)
    first = (t == 0)
    u_vec = jnp.where(first, u0_ref[...], u_scr[rslot])
    s_vec = jnp.where(first, s0_ref[...], s_scr[rslot])
    au = jnp.dot(a_ref[...], u_vec, preferred_element_type=jnp.float32)
    ps = jnp.dot(p_ref[...], s_vec, preferred_element_type=jnp.float32)
    s_new = coef_ref[t] * au + ps
    obase = i * _BM
    u_scr[wslot, pl.ds(obase, _BM), :] = au.astype(jnp.bfloat16)
    s_scr[wslot, pl.ds(obase, _BM), :] = s_new.astype(jnp.bfloat16)

    @pl.when(t == _NT - 1)
    def _():
        y_ref[...] = s_new


_step1 = pl.pallas_call(
    _step1_kern,
    grid=(_N // _BM1,),
    in_specs=[
        pl.BlockSpec((_BM1, _N), lambda i: (i, 0)),
        pl.BlockSpec((_BM1, _N), lambda i: (i, 0)),
        pl.BlockSpec((_N, _D), lambda i: (0, 0)),
        pl.BlockSpec((_N, _D), lambda i: (0, 0)),
    ],
    out_specs=[
        pl.BlockSpec((_BM1, _D), lambda i: (i, 0)),
        pl.BlockSpec((_BM1, _D), lambda i: (i, 0)),
        pl.BlockSpec((_BM1, _N), lambda i: (i, 0)),
        pl.BlockSpec((_BM1, _N), lambda i: (i, 0)),
    ],
    out_shape=[
        jax.ShapeDtypeStruct((_N, _D), jnp.float32),
        jax.ShapeDtypeStruct((_N, _D), jnp.float32),
        jax.ShapeDtypeStruct((_N, _N), jnp.bfloat16),
        jax.ShapeDtypeStruct((_N, _N), jnp.bfloat16),
    ],
    compiler_params=pltpu.CompilerParams(
        dimension_semantics=("arbitrary",),
    ),
)

_steps234 = pl.pallas_call(
    _steps_kern,
    grid=(_NT, _N // _BM),
    in_specs=[
        pl.BlockSpec(memory_space=pltpu.SMEM),
        pl.BlockSpec((_BM, _N), lambda t, i: (i, 0)),
        pl.BlockSpec((_BM, _N), lambda t, i: (i, 0)),
        pl.BlockSpec((_N, _D), lambda t, i: (0, 0)),
        pl.BlockSpec((_N, _D), lambda t, i: (0, 0)),
    ],
    out_specs=pl.BlockSpec((_BM, _D), lambda t, i: (i, 0)),
    out_shape=jax.ShapeDtypeStruct((_N, _D), jnp.float32),
    scratch_shapes=[
        pltpu.VMEM((2, _N, _D), jnp.bfloat16),
        pltpu.VMEM((2, _N, _D), jnp.bfloat16),
    ],
    compiler_params=pltpu.CompilerParams(
        dimension_semantics=("arbitrary", "arbitrary"),
    ),
)


def kernel(x, adj, poly_item, filter_param):
    k = filter_param.shape[0] - 1
    fp = jax.nn.relu(filter_param)[:, 0]
    coefs = [math.comb(k, i) / (2.0 ** k) for i in range(k + 1)]
    u = x
    s = coefs[k] * fp[k] * x
    au, ps, a16, p16 = _step1(adj, poly_item,
                              u.astype(jnp.bfloat16), s.astype(jnp.bfloat16))
    u16 = au.astype(jnp.bfloat16)
    s16 = (coefs[k - 1] * fp[k - 1] * au + ps).astype(jnp.bfloat16)
    cvec = jnp.stack([coefs[2] * fp[2], coefs[1] * fp[1], coefs[0] * fp[0]])
    y = _steps234(cvec, a16, p16, u16, s16)
    return y


# speedup vs baseline: 1.2123x; 1.0118x over previous
import math

import jax
import jax.numpy as jnp
from jax.experimental import pallas as pl
from jax.experimental.pallas import tpu as pltpu

_N = 8192
_D = 16
_BM = 512
_BM1 = 256
_NT = 3


def _step1_kern(coef_ref, a_ref, p_ref, u_ref, s_ref,
                u16_ref, s16_ref, a16_ref, p16_ref):
    a16 = a_ref[...].astype(jnp.bfloat16)
    p16 = p_ref[...].astype(jnp.bfloat16)
    a16_ref[...] = a16
    p16_ref[...] = p16
    au = jnp.dot(a16, u_ref[...], preferred_element_type=jnp.float32)
    ps = jnp.dot(p16, s_ref[...], preferred_element_type=jnp.float32)
    u16_ref[...] = au.astype(jnp.bfloat16)
    s16_ref[...] = (coef_ref[0] * au + ps).astype(jnp.bfloat16)


def _steps_kern(coef_ref, a_ref, p_ref, u0_ref, s0_ref, y_ref, u_scr, s_scr):
    t = pl.program_id(0)
    i = pl.program_id(1)

    @pl.when(jnp.logical_and(t == 0, i == 0))
    def _():
        u_scr[1] = u0_ref[...]
        s_scr[1] = s0_ref[...]

    rslot = jax.lax.rem(t + 1, 2)
    wslot = jax.lax.rem(t, 2)
    au = jnp.dot(a_ref[...], u_scr[rslot],
                 preferred_element_type=jnp.float32)
    ps = jnp.dot(p_ref[...], s_scr[rslot],
                 preferred_element_type=jnp.float32)
    s_new = coef_ref[t] * au + ps
    obase = i * _BM
    u_scr[wslot, pl.ds(obase, _BM), :] = au.astype(jnp.bfloat16)
    s_scr[wslot, pl.ds(obase, _BM), :] = s_new.astype(jnp.bfloat16)

    @pl.when(t == _NT - 1)
    def _():
        y_ref[...] = s_new


_step1 = pl.pallas_call(
    _step1_kern,
    grid=(_N // _BM1,),
    in_specs=[
        pl.BlockSpec(memory_space=pltpu.SMEM),
        pl.BlockSpec((_BM1, _N), lambda i: (i, 0)),
        pl.BlockSpec((_BM1, _N), lambda i: (i, 0)),
        pl.BlockSpec((_N, _D), lambda i: (0, 0)),
        pl.BlockSpec((_N, _D), lambda i: (0, 0)),
    ],
    out_specs=[
        pl.BlockSpec((_BM1, _D), lambda i: (i, 0)),
        pl.BlockSpec((_BM1, _D), lambda i: (i, 0)),
        pl.BlockSpec((_BM1, _N), lambda i: (i, 0)),
        pl.BlockSpec((_BM1, _N), lambda i: (i, 0)),
    ],
    out_shape=[
        jax.ShapeDtypeStruct((_N, _D), jnp.bfloat16),
        jax.ShapeDtypeStruct((_N, _D), jnp.bfloat16),
        jax.ShapeDtypeStruct((_N, _N), jnp.bfloat16),
        jax.ShapeDtypeStruct((_N, _N), jnp.bfloat16),
    ],
    compiler_params=pltpu.CompilerParams(
        dimension_semantics=("arbitrary",),
    ),
)

_steps234 = pl.pallas_call(
    _steps_kern,
    grid=(_NT, _N // _BM),
    in_specs=[
        pl.BlockSpec(memory_space=pltpu.SMEM),
        pl.BlockSpec((_BM, _N), lambda t, i: (i, 0)),
        pl.BlockSpec((_BM, _N), lambda t, i: (i, 0)),
        pl.BlockSpec((_N, _D), lambda t, i: (0, 0)),
        pl.BlockSpec((_N, _D), lambda t, i: (0, 0)),
    ],
    out_specs=pl.BlockSpec((_BM, _D), lambda t, i: (i, 0)),
    out_shape=jax.ShapeDtypeStruct((_N, _D), jnp.float32),
    scratch_shapes=[
        pltpu.VMEM((2, _N, _D), jnp.bfloat16),
        pltpu.VMEM((2, _N, _D), jnp.bfloat16),
    ],
    compiler_params=pltpu.CompilerParams(
        dimension_semantics=("arbitrary", "arbitrary"),
    ),
)


def kernel(x, adj, poly_item, filter_param):
    k = filter_param.shape[0] - 1
    fp = jax.nn.relu(filter_param)[:, 0]
    coefs = [math.comb(k, i) / (2.0 ** k) for i in range(k + 1)]
    c3 = (coefs[k - 1] * fp[k - 1]).reshape(1)
    u16, s16, a16, p16 = _step1(c3, adj, poly_item,
                                x.astype(jnp.bfloat16),
                                (coefs[k] * fp[k] * x).astype(jnp.bfloat16))
    cvec = jnp.stack([coefs[2] * fp[2], coefs[1] * fp[1], coefs[0] * fp[0]])
    y = _steps234(cvec, a16, p16, u16, s16)
    return y
